# Initial kernel scaffold; baseline (speedup 1.0000x reference)
import jax
import jax.numpy as jnp
from jax.experimental import pallas as pl


def kernel(x, time_emb):
    raise NotImplementedError("write your pallas kernel here")


# speedup vs baseline: 3.5886x; 3.5886x over previous
import dataclasses
import functools

import jax
import jax.numpy as jnp
from jax import lax
from jax.experimental import pallas as pl
from jax.experimental.pallas import tpu as pltpu
from jax.experimental.pallas import tpu_sc as plsc

DIM = 64
MAX_T = 72
LANES = 16
NC, NS = 2, 16
NW = NC * NS
CH = 128


def _build_pair_table(time_emb):
    def body(row_ref, all_ref, o_ref):
        o_ref[0, :, 0:DIM] = jnp.broadcast_to(row_ref[0], (MAX_T, DIM))
        o_ref[0, :, DIM:2 * DIM] = all_ref[...]

    emb3 = time_emb.reshape(MAX_T, 1, DIM)
    pt3 = pl.pallas_call(
        body,
        grid=(MAX_T,),
        in_specs=[
            pl.BlockSpec((1, 1, DIM), lambda i: (i, 0, 0)),
            pl.BlockSpec((MAX_T, DIM), lambda i: (0, 0)),
        ],
        out_specs=pl.BlockSpec((1, MAX_T, 2 * DIM), lambda i: (i, 0, 0)),
        out_shape=jax.ShapeDtypeStruct((MAX_T, MAX_T, 2 * DIM), jnp.float32),
    )(emb3, time_emb)
    return pt3.reshape(MAX_T * MAX_T, 2 * DIM)


def kernel(x, time_emb):
    B, L = x.shape
    N = B * L
    NP = N // 2
    per_w = NP // NW
    n_chunks = per_w // CH
    idx = x.reshape(N)

    pair_table = _build_pair_table(time_emb)

    mesh = plsc.VectorSubcoreMesh(core_axis_name="c", subcore_axis_name="s")

    cp = pltpu.CompilerParams()
    if "needs_layout_passes" in pltpu.CompilerParams.__dataclass_fields__:
        cp = dataclasses.replace(cp, needs_layout_passes=False)

    @functools.partial(
        pl.kernel,
        mesh=mesh,
        compiler_params=cp,
        out_type=jax.ShapeDtypeStruct((NP, 2 * DIM), jnp.float32),
        scratch_types=[
            pltpu.VMEM((2 * CH,), jnp.int32),
            pltpu.VMEM((CH,), jnp.int32),
            pltpu.VMEM((CH, 2 * DIM), jnp.float32),
            pltpu.SemaphoreType.DMA,
        ],
    )
    def k(pt_hbm, idx_hbm, out_hbm, raw_v, comb_v, rows_v, sem):
        wid = lax.axis_index("s") * NC + lax.axis_index("c")
        base0 = wid * per_w

        @pl.loop(0, n_chunks)
        def _(g):
            base = base0 + g * CH
            pltpu.sync_copy(idx_hbm.at[pl.ds(2 * base, 2 * CH)], raw_v)

            @pl.loop(0, CH, step=LANES)
            def _(c):
                pos = 2 * (c + lax.iota(jnp.int32, LANES))
                ev = plsc.load_gather(raw_v, [pos])
                od = plsc.load_gather(raw_v, [pos + 1])
                ev = jnp.minimum(jnp.maximum(ev, 0), MAX_T - 1)
                od = jnp.minimum(jnp.maximum(od, 0), MAX_T - 1)
                comb_v[pl.ds(c, LANES)] = ev * MAX_T + od

            pltpu.async_copy(pt_hbm.at[comb_v], rows_v, sem).wait()
            pltpu.sync_copy(rows_v, out_hbm.at[pl.ds(base, CH)])

    out = k(pair_table, idx)
    return out.reshape(B, L, DIM)


# speedup vs baseline: 4.1961x; 1.1693x over previous
import dataclasses
import functools

import jax
import jax.numpy as jnp
from jax import lax
from jax.experimental import pallas as pl
from jax.experimental.pallas import tpu as pltpu
from jax.experimental.pallas import tpu_sc as plsc

DIM = 64
MAX_T = 72
LANES = 16
NC, NS = 2, 16
NW = NC * NS
CH = 128
K = 4


def _build_pair_table(time_emb):
    def body(row_ref, all_ref, o_ref):
        o_ref[0, :, 0:DIM] = jnp.broadcast_to(row_ref[0], (MAX_T, DIM))
        o_ref[0, :, DIM:2 * DIM] = all_ref[...]

    emb3 = time_emb.reshape(MAX_T, 1, DIM)
    pt3 = pl.pallas_call(
        body,
        grid=(MAX_T,),
        in_specs=[
            pl.BlockSpec((1, 1, DIM), lambda i: (i, 0, 0)),
            pl.BlockSpec((MAX_T, DIM), lambda i: (0, 0)),
        ],
        out_specs=pl.BlockSpec((1, MAX_T, 2 * DIM), lambda i: (i, 0, 0)),
        out_shape=jax.ShapeDtypeStruct((MAX_T, MAX_T, 2 * DIM), jnp.float32),
    )(emb3, time_emb)
    return pt3.reshape(MAX_T * MAX_T, 2 * DIM)


def kernel(x, time_emb):
    B, L = x.shape
    N = B * L
    NP = N // 2
    per_w = NP // NW
    n_chunks = per_w // CH
    idx = x.reshape(NW, 2 * per_w)

    pair_table = _build_pair_table(time_emb)

    mesh = plsc.VectorSubcoreMesh(core_axis_name="c", subcore_axis_name="s")

    cp = pltpu.CompilerParams()
    if "needs_layout_passes" in pltpu.CompilerParams.__dataclass_fields__:
        cp = dataclasses.replace(cp, needs_layout_passes=False)

    @functools.partial(
        pl.kernel,
        mesh=mesh,
        compiler_params=cp,
        out_type=jax.ShapeDtypeStruct((NP, 2 * DIM), jnp.float32),
        scratch_types=[
            pltpu.VMEM((2 * per_w,), jnp.int32),
            pltpu.VMEM((n_chunks, CH), jnp.int32),
            pltpu.VMEM((K, CH, 2 * DIM), jnp.float32),
        ]
        + [pltpu.SemaphoreType.DMA] * (2 * K),
    )
    def k(pt_hbm, idx_hbm, out_hbm, raw_v, comb_v, rows_v, *sems):
        gsems, osems = sems[:K], sems[K:]
        wid = lax.axis_index("s") * NC + lax.axis_index("c")
        base0 = wid * per_w

        pltpu.sync_copy(idx_hbm.at[wid], raw_v)

        @pl.loop(0, n_chunks)
        def _(r):
            @pl.loop(0, CH, step=LANES)
            def _(c):
                pos = 2 * (r * CH + c + lax.iota(jnp.int32, LANES))
                ev = plsc.load_gather(raw_v, [pos])
                od = plsc.load_gather(raw_v, [pos + 1])
                ev = jnp.minimum(jnp.maximum(ev, 0), MAX_T - 1)
                od = jnp.minimum(jnp.maximum(od, 0), MAX_T - 1)
                comb_v[r, pl.ds(c, LANES)] = ev * MAX_T + od

        @pl.loop(0, n_chunks, step=K)
        def _(g0):
            gathers = []
            for p in range(K):
                g = g0 + p

                @pl.when(g0 != 0)
                def _():
                    pltpu.make_async_copy(
                        rows_v.at[p],
                        out_hbm.at[pl.ds(base0 + (g - K) * CH, CH)],
                        osems[p],
                    ).wait()

                gathers.append(
                    pltpu.async_copy(
                        pt_hbm.at[comb_v.at[g]], rows_v.at[p], gsems[p]
                    )
                )
            for p in range(K):
                gathers[p].wait()
                pltpu.async_copy(
                    rows_v.at[p],
                    out_hbm.at[pl.ds(base0 + (g0 + p) * CH, CH)],
                    osems[p],
                )

        for p in range(K):
            pltpu.make_async_copy(
                rows_v.at[p],
                out_hbm.at[pl.ds(base0 + (n_chunks - K + p) * CH, CH)],
                osems[p],
            ).wait()

    out = k(pair_table, idx)
    return out.reshape(B, L, DIM)
